# baseline (speedup 1.0000x reference)
import functools

import jax
import jax.numpy as jnp
from jax import lax
from jax.experimental import pallas as pl
from jax.experimental.pallas import tpu as pltpu
from jax.experimental.pallas import tpu_sc as plsc

DIM = 64
NEG = 10
NC = 2
NS = 16
NW = NC * NS
LANES = 16
CHUNK = 64
SROW = 2 * DIM


def _sc_scores(u_w2, v_w2, u_pos, v_pos, v_neg_flat, batch):
    bpw = batch // NW
    nchunks = bpw // CHUNK
    mesh = plsc.VectorSubcoreMesh(
        core_axis_name="c", subcore_axis_name="s", num_cores=NC, num_subcores=NS
    )

    @functools.partial(
        pl.kernel,
        out_type=[
            jax.ShapeDtypeStruct((batch,), jnp.float32),
            jax.ShapeDtypeStruct((batch,), jnp.float32),
        ],
        mesh=mesh,
        compiler_params=pltpu.CompilerParams(needs_layout_passes=False),
        scratch_types=[
            pltpu.VMEM((CHUNK,), jnp.int32),
            pltpu.VMEM((CHUNK,), jnp.int32),
            pltpu.VMEM((NEG * CHUNK,), jnp.int32),
            pltpu.VMEM((CHUNK,), jnp.int32),
            pltpu.VMEM((CHUNK,), jnp.int32),
            pltpu.VMEM((NEG * CHUNK,), jnp.int32),
            pltpu.VMEM((CHUNK, SROW), jnp.float32),
            pltpu.VMEM((CHUNK, SROW), jnp.float32),
            pltpu.VMEM((NEG * CHUNK, SROW), jnp.float32),
            pltpu.VMEM((CHUNK,), jnp.float32),
            pltpu.VMEM((CHUNK,), jnp.float32),
            pltpu.SemaphoreType.DMA,
        ],
    )
    def sc_kernel(u_w, v_w, up, vp, vn, score_out, nscore_out,
                  idx_u, idx_v, idx_n, div_u, div_v, div_n,
                  rows_u, rows_v, rows_n, sc_chunk, nc_chunk, sem):
        wid = lax.axis_index("s") * NC + lax.axis_index("c")
        base = wid * bpw
        lane_iota = lax.iota(jnp.int32, LANES)
        for c in range(nchunks):
            off = base + c * CHUNK
            pltpu.sync_copy(up.at[pl.ds(off, CHUNK)], idx_u)
            pltpu.sync_copy(vp.at[pl.ds(off, CHUNK)], idx_v)
            pltpu.sync_copy(vn.at[pl.ds(off * NEG, CHUNK * NEG)], idx_n)
            for t in range(CHUNK // LANES):
                sl = pl.ds(t * LANES, LANES)
                div_u[sl] = idx_u[sl] >> 1
                div_v[sl] = idx_v[sl] >> 1
            for t in range(NEG * CHUNK // LANES):
                sl = pl.ds(t * LANES, LANES)
                div_n[sl] = idx_n[sl] >> 1
            cps = [
                pltpu.async_copy(u_w.at[div_u], rows_u, sem),
                pltpu.async_copy(v_w.at[div_v], rows_v, sem),
            ]
            for j in range(NEG):
                cps.append(
                    pltpu.async_copy(
                        v_w.at[div_n.at[pl.ds(j * CHUNK, CHUNK)]],
                        rows_n.at[pl.ds(j * CHUNK, CHUNK)],
                        sem,
                    )
                )
            for cp in cps:
                cp.wait()

            for g in range(CHUNK // LANES):
                gsl = pl.ds(g * LANES, LANES)
                rowp = g * LANES + lane_iota
                cu = (idx_u[gsl] & 1) * DIM
                cv = (idx_v[gsl] & 1) * DIM
                nrow = []
                ncol = []
                for n in range(NEG):
                    ni = plsc.load_gather(
                        idx_n, [lane_iota * NEG + (g * LANES * NEG + n)])
                    nrow.append(rowp * NEG + n)
                    ncol.append((ni & 1) * DIM)

                def dloop(d, carry):
                    acc_s, acc_n = carry
                    gu = plsc.load_gather(rows_u, [rowp, cu + d])
                    gv = plsc.load_gather(rows_v, [rowp, cv + d])
                    gn = None
                    for n in range(NEG):
                        gx = plsc.load_gather(rows_n, [nrow[n], ncol[n] + d])
                        gn = gx if gn is None else gn + gx
                    return (acc_s + gu * gv, acc_n + gu * gn)

                zeros = jnp.zeros((LANES,), jnp.float32)
                acc_s, acc_n = lax.fori_loop(0, DIM, dloop, (zeros, zeros))
                sc_chunk[gsl] = acc_s
                nc_chunk[gsl] = acc_n

            pltpu.sync_copy(sc_chunk, score_out.at[pl.ds(off, CHUNK)])
            pltpu.sync_copy(nc_chunk, nscore_out.at[pl.ds(off, CHUNK)])

    return sc_kernel(u_w2, v_w2, u_pos, v_pos, v_neg_flat)


def _tc_loss_body(s_ref, n_ref, o_ref):
    s = s_ref[...]
    n = n_ref[...]
    val = jax.nn.log_sigmoid(s) + jax.nn.log_sigmoid(-n)
    o_ref[0, 0] = -jnp.sum(val)


def kernel(u_pos, v_pos, v_neg, batch_size, u_weight, v_weight):
    batch = u_pos.shape[0]
    vocab = u_weight.shape[0]
    u_w2 = u_weight.reshape(vocab // 2, SROW)
    v_w2 = v_weight.reshape(vocab // 2, SROW)
    scores, nscores = _sc_scores(
        u_w2,
        v_w2,
        u_pos.astype(jnp.int32),
        v_pos.astype(jnp.int32),
        v_neg.reshape(-1).astype(jnp.int32),
        batch,
    )
    rows = batch // 128
    loss_sum = pl.pallas_call(
        _tc_loss_body,
        out_shape=jax.ShapeDtypeStruct((1, 1), jnp.float32),
        out_specs=pl.BlockSpec(memory_space=pltpu.SMEM),
    )(scores.reshape(rows, 128), nscores.reshape(rows, 128))
    return loss_sum[0, 0] / batch_size
